# baseline (speedup 1.0000x reference)
import functools

import jax
import jax.numpy as jnp
from jax import lax
from jax.experimental import pallas as pl
from jax.experimental.pallas import tpu as pltpu
from jax.experimental.pallas import tpu_sc as plsc

_B, _L, _D = 4, 2048, 1024
_NC, _NS = 2, 16
_NW = _NC * _NS
_LPW = _L // _NW
_CH_ROWS = 32
_NCH = _LPW // _CH_ROWS
_CHW = _CH_ROWS * _D

_mesh = plsc.VectorSubcoreMesh(
    core_axis_name="c", subcore_axis_name="s", num_cores=_NC, num_subcores=_NS
)


@functools.partial(
    pl.kernel,
    out_type=jax.ShapeDtypeStruct((_B * _L * _D,), jnp.float32),
    mesh=_mesh,
    scratch_types=[
        pltpu.VMEM((_CHW,), jnp.float32),
        pltpu.VMEM((_CHW,), jnp.float32),
    ],
)
def _pos_add(x_hbm, pos_hbm, out_hbm, pos_v, x_v):
    wid = lax.axis_index("s") * _NC + lax.axis_index("c")
    lbase = wid * _LPW
    for lc in range(_NCH):
        pstart = (lbase + lc * _CH_ROWS) * _D
        pltpu.sync_copy(pos_hbm.at[pl.ds(pstart, _CHW)], pos_v)
        for b in range(_B):
            xstart = b * _L * _D + pstart
            pltpu.sync_copy(x_hbm.at[pl.ds(xstart, _CHW)], x_v)

            @plsc.parallel_loop(0, _CHW, step=16, unroll=8)
            def _(i):
                x_v[pl.ds(i, 16)] = x_v[pl.ds(i, 16)] + pos_v[pl.ds(i, 16)]

            pltpu.sync_copy(x_v, out_hbm.at[pl.ds(xstart, _CHW)])


def kernel(x, pos_table):
    out = _pos_add(x.reshape(-1), pos_table.reshape(-1))
    return out.reshape(x.shape)


# speedup vs baseline: 1.1185x; 1.1185x over previous
import functools

import jax
import jax.numpy as jnp
from jax import lax
from jax.experimental import pallas as pl
from jax.experimental.pallas import tpu as pltpu
from jax.experimental.pallas import tpu_sc as plsc

_B, _L, _D = 4, 2048, 1024
_NC, _NS = 2, 16
_NW = _NC * _NS
_LPW = _L // _NW
_POSW = _LPW * _D
_CH_ROWS = 16
_NLC = _LPW // _CH_ROWS
_CHW = _CH_ROWS * _D
_NCHUNK = _NLC * _B

_mesh = plsc.VectorSubcoreMesh(
    core_axis_name="c", subcore_axis_name="s", num_cores=_NC, num_subcores=_NS
)


@functools.partial(
    pl.kernel,
    out_type=jax.ShapeDtypeStruct((_B * _L * _D,), jnp.float32),
    mesh=_mesh,
    scratch_types=[
        pltpu.VMEM((_POSW,), jnp.float32),
        pltpu.VMEM((_CHW,), jnp.float32),
        pltpu.VMEM((_CHW,), jnp.float32),
        pltpu.SemaphoreType.DMA,
        pltpu.SemaphoreType.DMA,
        pltpu.SemaphoreType.DMA,
        pltpu.SemaphoreType.DMA,
        pltpu.SemaphoreType.DMA,
    ],
)
def _pos_add(x_hbm, pos_hbm, out_hbm, pos_v, xa, xb,
             pos_sem, in0, in1, out0, out1):
    wid = lax.axis_index("s") * _NC + lax.axis_index("c")
    lbase = wid * _LPW * _D
    bufs = (xa, xb)
    in_sems = (in0, in1)
    out_sems = (out0, out1)

    def x_off(k):
        lc, b = divmod(k, _B)
        return b * _L * _D + lbase + lc * _CHW

    pos_cp = pltpu.make_async_copy(pos_hbm.at[pl.ds(lbase, _POSW)], pos_v,
                                   pos_sem)
    pos_cp.start()

    loads = [
        pltpu.make_async_copy(x_hbm.at[pl.ds(x_off(k), _CHW)], bufs[k % 2],
                              in_sems[k % 2])
        for k in range(_NCHUNK)
    ]
    stores = [
        pltpu.make_async_copy(bufs[k % 2], out_hbm.at[pl.ds(x_off(k), _CHW)],
                              out_sems[k % 2])
        for k in range(_NCHUNK)
    ]

    loads[0].start()
    for k in range(_NCHUNK):
        if k + 1 < _NCHUNK:
            if k >= 1:
                stores[k - 1].wait()
            loads[k + 1].start()
        loads[k].wait()
        if k == 0:
            pos_cp.wait()
        x_v = bufs[k % 2]
        poff = (k // _B) * _CHW

        @plsc.parallel_loop(0, _CHW, step=16, unroll=8)
        def _(i):
            plsc.addupdate(x_v.at[pl.ds(i, 16)], pos_v[pl.ds(poff + i, 16)])

        stores[k].start()
    stores[_NCHUNK - 2].wait()
    stores[_NCHUNK - 1].wait()


def kernel(x, pos_table):
    out = _pos_add(x.reshape(-1), pos_table.reshape(-1))
    return out.reshape(x.shape)


# speedup vs baseline: 2.4334x; 2.1756x over previous
import functools

import jax
import jax.numpy as jnp
from jax import lax
from jax.experimental import pallas as pl
from jax.experimental.pallas import tpu as pltpu
from jax.experimental.pallas import tpu_sc as plsc

_B, _L, _D = 4, 2048, 1024
_NC, _NS = 2, 16
_NW = _NC * _NS
_LPW = _L // _NW
_CH = 16
_NLC = _LPW // _CH
_NCHUNK = _NLC * _B

_mesh = plsc.VectorSubcoreMesh(
    core_axis_name="c", subcore_axis_name="s", num_cores=_NC, num_subcores=_NS
)


@functools.partial(
    pl.kernel,
    out_type=jax.ShapeDtypeStruct((_B * _L, _D), jnp.float32),
    mesh=_mesh,
    scratch_types=[
        pltpu.VMEM((_LPW, _D), jnp.float32),
        pltpu.VMEM((_CH, _D), jnp.float32),
        pltpu.VMEM((_CH, _D), jnp.float32),
        pltpu.SemaphoreType.DMA,
        pltpu.SemaphoreType.DMA,
        pltpu.SemaphoreType.DMA,
        pltpu.SemaphoreType.DMA,
        pltpu.SemaphoreType.DMA,
    ],
)
def _pos_add(x_hbm, pos_hbm, out_hbm, pos_v, xa, xb,
             pos_sem, in0, in1, out0, out1):
    wid = lax.axis_index("s") * _NC + lax.axis_index("c")
    lbase = wid * _LPW
    bufs = (xa, xb)
    in_sems = (in0, in1)
    out_sems = (out0, out1)

    def x_row(k):
        lc, b = divmod(k, _B)
        return b * _L + lbase + lc * _CH

    pos_cp = pltpu.make_async_copy(pos_hbm.at[pl.ds(lbase, _LPW), :], pos_v,
                                   pos_sem)
    pos_cp.start()

    loads = [
        pltpu.make_async_copy(x_hbm.at[pl.ds(x_row(k), _CH), :], bufs[k % 2],
                              in_sems[k % 2])
        for k in range(_NCHUNK)
    ]
    stores = [
        pltpu.make_async_copy(bufs[k % 2], out_hbm.at[pl.ds(x_row(k), _CH), :],
                              out_sems[k % 2])
        for k in range(_NCHUNK)
    ]

    loads[0].start()
    for k in range(_NCHUNK):
        if k + 1 < _NCHUNK:
            if k >= 1:
                stores[k - 1].wait()
            loads[k + 1].start()
        loads[k].wait()
        if k == 0:
            pos_cp.wait()
        x_v = bufs[k % 2]
        prow = (k // _B) * _CH

        @plsc.parallel_loop(0, _D, step=16, unroll=2)
        def _(i):
            for r in range(_CH):
                plsc.addupdate(x_v.at[r, pl.ds(i, 16)],
                               pos_v[prow + r, pl.ds(i, 16)])

        stores[k].start()
    stores[_NCHUNK - 2].wait()
    stores[_NCHUNK - 1].wait()


def kernel(x, pos_table):
    out = _pos_add(x.reshape(_B * _L, _D), pos_table)
    return out.reshape(x.shape)
